# baseline (speedup 1.0000x reference)
import jax
import jax.numpy as jnp
from jax.experimental import pallas as pl
from jax.experimental.pallas import tpu as pltpu

B, N, F, S, C, NOUT = 32, 32, 32, 4, 32, 16


def _fold_machinery():
    f32 = jnp.float32
    row = jax.lax.broadcasted_iota(jnp.int32, (F, F * C), 0)
    col = jax.lax.broadcasted_iota(jnp.int32, (F, F * C), 1)
    blk_mask = (col // C == row).astype(f32)
    mrow = jax.lax.broadcasted_iota(jnp.int32, (F * C, C), 0)
    mcol = jax.lax.broadcasted_iota(jnp.int32, (F * C, C), 1)
    collapse = (mrow % C == mcol).astype(f32)
    return blk_mask, collapse


def _fold_row(w_row, blk_mask, collapse):
    rep = jnp.broadcast_to(w_row, (F, F * C))
    return jnp.dot(rep * blk_mask, collapse, preferred_element_type=jnp.float32)


def _net_kernel(x2_ref, a2_ref, e2_ref, w1_ref, wb1_ref, root1_ref, b1_ref,
                w2_ref, wb2_ref, root2_ref, b2_ref, dw_ref, db_ref,
                out_ref, ae_s, tu_s, r_s, h_s):
    f32 = jnp.float32
    blk_mask, collapse = _fold_machinery()
    feats = x2_ref[:, :F]

    prow = jax.lax.broadcasted_iota(jnp.int32, (N * S, S * N), 0)
    pcol = jax.lax.broadcasted_iota(jnp.int32, (N * S, S * N), 1)
    perm = ((prow % S) * N + prow // S == pcol).astype(jnp.bfloat16)
    et = jnp.dot(e2_ref[:], perm, preferred_element_type=f32)

    a2 = a2_ref[:]
    a4 = jnp.concatenate([a2, a2, a2, a2], axis=1)
    ae_s[:] = jnp.concatenate([a4 * et, a2], axis=1)

    def node_stage(src, w_ref, wb_ref, root_ref, b_ref):
        blocks = [jnp.dot(src, _fold_row(w_ref[s:s + 1, :], blk_mask, collapse),
                          preferred_element_type=f32) for s in range(S)]
        blocks.append(jnp.dot(src, _fold_row(wb_ref[:], blk_mask, collapse),
                              preferred_element_type=f32))
        tu_s[:] = jnp.concatenate(blocks, axis=1)
        r_s[:] = (jnp.dot(src, root_ref[:], preferred_element_type=f32)
                  + jnp.reshape(b_ref[:], (1, C)))

    def conv_rows(b):
        ae = ae_s[b * N:(b + 1) * N, :]
        tb = tu_s[b * N:(b + 1) * N, :]
        tu = jnp.concatenate(
            [tb[:, 0:C], tb[:, C:2 * C], tb[:, 2 * C:3 * C], tb[:, 3 * C:4 * C],
             tb[:, 4 * C:5 * C]], axis=0)
        rb = r_s[b * N:(b + 1) * N, :]
        return jnp.maximum(jnp.dot(ae, tu, preferred_element_type=f32) + rb, 0.0)

    node_stage(feats, w1_ref, wb1_ref, root1_ref, b1_ref)
    for b in range(B):
        h_s[b * N:(b + 1) * N, :] = conv_rows(b)

    node_stage(h_s[:], w2_ref, wb2_ref, root2_ref, b2_ref)

    mcol = (x2_ref[:, F:F + 1] != 0.0).astype(f32)
    rows = []
    for b in range(B):
        h2 = conv_rows(b)
        mb = mcol[b * N:(b + 1) * N, :]
        rows.append(jnp.sum(h2 * mb, axis=0, keepdims=True))
    pooled = jnp.concatenate(rows, axis=0)
    out_ref[:] = (jnp.dot(pooled, dw_ref[:], preferred_element_type=f32)
                  + jnp.reshape(db_ref[:], (1, NOUT)))


def kernel(x, a, e, kn_w1, kn_b1, root1, bias1, kn_w2, kn_b2, root2, bias2,
           dense_w, dense_b):
    f32 = jnp.float32
    x2 = x.reshape(B * N, F + 1)
    a2 = a.reshape(B * N, N)
    e2 = e.astype(jnp.bfloat16).reshape(B * N, N * S)
    wb1 = kn_b1.reshape(1, F * C)
    wb2 = kn_b2.reshape(1, C * C)

    return pl.pallas_call(
        _net_kernel,
        out_shape=jax.ShapeDtypeStruct((B, NOUT), f32),
        scratch_shapes=[
            pltpu.VMEM((B * N, S * N + N), f32),
            pltpu.VMEM((B * N, (S + 1) * C), f32),
            pltpu.VMEM((B * N, C), f32),
            pltpu.VMEM((B * N, C), f32),
        ],
    )(x2, a2, e2, kn_w1, wb1, root1, bias1,
      kn_w2, wb2, root2, bias2, dense_w, dense_b)


# speedup vs baseline: 1.0699x; 1.0699x over previous
import jax
import jax.numpy as jnp
from jax.experimental import pallas as pl
from jax.experimental.pallas import tpu as pltpu

B, N, F, S, C, NOUT = 32, 32, 32, 4, 32, 16


def _fold_machinery():
    f32 = jnp.float32
    row = jax.lax.broadcasted_iota(jnp.int32, (F, F * C), 0)
    col = jax.lax.broadcasted_iota(jnp.int32, (F, F * C), 1)
    blk_mask = (col // C == row).astype(f32)
    mrow = jax.lax.broadcasted_iota(jnp.int32, (F * C, C), 0)
    mcol = jax.lax.broadcasted_iota(jnp.int32, (F * C, C), 1)
    collapse = (mrow % C == mcol).astype(f32)
    return blk_mask, collapse


def _fold_row(w_row, blk_mask, collapse):
    rep = jnp.broadcast_to(w_row, (F, F * C))
    return jnp.dot(rep * blk_mask, collapse, preferred_element_type=jnp.float32)


def _net_kernel(x2_ref, a2_ref, e2_ref, w1_ref, wb1_ref, root1_ref, b1_ref,
                w2_ref, wb2_ref, root2_ref, b2_ref, dw_ref, db_ref,
                out_ref, ae_s, tu_s, r_s, h_s):
    f32 = jnp.float32
    blk_mask, collapse = _fold_machinery()
    feats = x2_ref[:, :F]

    prow = jax.lax.broadcasted_iota(jnp.int32, (N * S, S * N), 0)
    pcol = jax.lax.broadcasted_iota(jnp.int32, (N * S, S * N), 1)
    perm = ((prow % S) * N + prow // S == pcol).astype(f32)
    et = jnp.dot(e2_ref[:], perm, preferred_element_type=f32)

    a2 = a2_ref[:]
    a4 = jnp.concatenate([a2, a2, a2, a2], axis=1)
    ae_s[:] = jnp.concatenate([a4 * et, a2], axis=1)

    def node_stage(src, w_ref, wb_ref, root_ref, b_ref):
        blocks = [jnp.dot(src, _fold_row(w_ref[s:s + 1, :], blk_mask, collapse),
                          preferred_element_type=f32) for s in range(S)]
        blocks.append(jnp.dot(src, _fold_row(wb_ref[:], blk_mask, collapse),
                              preferred_element_type=f32))
        tu_s[:] = jnp.concatenate(blocks, axis=1)
        r_s[:] = (jnp.dot(src, root_ref[:], preferred_element_type=f32)
                  + jnp.reshape(b_ref[:], (1, C)))

    def conv_rows(b):
        ae = ae_s[b * N:(b + 1) * N, :]
        tb = tu_s[b * N:(b + 1) * N, :]
        tu = jnp.concatenate(
            [tb[:, 0:C], tb[:, C:2 * C], tb[:, 2 * C:3 * C], tb[:, 3 * C:4 * C],
             tb[:, 4 * C:5 * C]], axis=0)
        rb = r_s[b * N:(b + 1) * N, :]
        return jnp.maximum(jnp.dot(ae, tu, preferred_element_type=f32) + rb, 0.0)

    node_stage(feats, w1_ref, wb1_ref, root1_ref, b1_ref)
    for b in range(B):
        h_s[b * N:(b + 1) * N, :] = conv_rows(b)

    node_stage(h_s[:], w2_ref, wb2_ref, root2_ref, b2_ref)

    mcol = (x2_ref[:, F:F + 1] != 0.0).astype(f32)
    rows = []
    for b in range(B):
        h2 = conv_rows(b)
        mb = mcol[b * N:(b + 1) * N, :]
        rows.append(jnp.sum(h2 * mb, axis=0, keepdims=True))
    pooled = jnp.concatenate(rows, axis=0)
    out_ref[:] = (jnp.dot(pooled, dw_ref[:], preferred_element_type=f32)
                  + jnp.reshape(db_ref[:], (1, NOUT)))


def kernel(x, a, e, kn_w1, kn_b1, root1, bias1, kn_w2, kn_b2, root2, bias2,
           dense_w, dense_b):
    f32 = jnp.float32
    x2 = x.reshape(B * N, F + 1)
    a2 = a.reshape(B * N, N)
    e2 = e.reshape(B * N, N * S)
    wb1 = kn_b1.reshape(1, F * C)
    wb2 = kn_b2.reshape(1, C * C)

    return pl.pallas_call(
        _net_kernel,
        out_shape=jax.ShapeDtypeStruct((B, NOUT), f32),
        compiler_params=pltpu.CompilerParams(
            allow_input_fusion=[True] * 13),
        scratch_shapes=[
            pltpu.VMEM((B * N, S * N + N), f32),
            pltpu.VMEM((B * N, (S + 1) * C), f32),
            pltpu.VMEM((B * N, C), f32),
            pltpu.VMEM((B * N, C), f32),
        ],
    )(x2, a2, e2, kn_w1, wb1, root1, bias1,
      kn_w2, wb2, root2, bias2, dense_w, dense_b)
